# baseline (speedup 1.0000x reference)
import functools

import jax
import jax.numpy as jnp
from jax import lax
from jax.experimental import pallas as pl
from jax.experimental.pallas import tpu as pltpu
from jax.experimental.pallas import tpu_sc as plsc

_CH = 128


@functools.lru_cache(maxsize=None)
def _make_gather(B, D, num_cores, num_subcores):
    NW = num_cores * num_subcores
    b_per_w = B // NW
    n_ch = b_per_w // _CH
    mesh = plsc.VectorSubcoreMesh(core_axis_name="c", subcore_axis_name="s")

    @functools.partial(
        pl.kernel,
        mesh=mesh,
        out_type=jax.ShapeDtypeStruct((B, D), jnp.float32),
        scratch_types=[
            pltpu.VMEM((n_ch, _CH), jnp.int32),
            pltpu.VMEM((2, _CH, D), jnp.float32),
            pltpu.SemaphoreType.DMA,
            pltpu.SemaphoreType.DMA,
        ],
        compiler_params=pltpu.CompilerParams(use_tc_tiling_on_sc=False),
    )
    def body(idx_hbm, w_hbm, out_hbm, idx_v, rows_v, gsem0, gsem1):
        wid = lax.axis_index("s") * num_cores + lax.axis_index("c")
        base_ch = wid * n_ch
        pltpu.sync_copy(idx_hbm.at[pl.ds(base_ch, n_ch)], idx_v)

        def gather(j, slot, sem):
            return pltpu.make_async_copy(
                w_hbm.at[idx_v.at[j]], rows_v.at[slot], sem
            )

        def step(j):
            gather(j, 0, gsem0).start()
            gather(j, 0, gsem0).wait()
            pltpu.sync_copy(
                rows_v.at[0], out_hbm.at[pl.ds((base_ch + j) * _CH, _CH)]
            )

        pl.loop(0, n_ch)(step)

    return body


def kernel(x, weight):
    B_, H = x.shape
    V, D = weight.shape
    B = B_ * H
    info = plsc.get_sparse_core_info()
    idx = x.reshape(B // _CH, _CH)
    fn = _make_gather(B, D, info.num_cores, info.num_subcores)
    out = fn(idx, weight)
    return out.reshape(B_, H, D)
